# Initial kernel scaffold; baseline (speedup 1.0000x reference)
import jax
import jax.numpy as jnp
from jax.experimental import pallas as pl


def kernel(x, Wa, Wb):
    raise NotImplementedError("write your pallas kernel here")


# speedup vs baseline: 1.5204x; 1.5204x over previous
import functools

import jax
import jax.numpy as jnp
from jax import lax
from jax.experimental import pallas as pl
from jax.experimental.pallas import tpu as pltpu
from jax.experimental.pallas import tpu_sc as plsc

_NUM_CORES = 2
_NUM_SUBCORES = 16
_NUM_WORKERS = _NUM_CORES * _NUM_SUBCORES
_LANES = 16

_SCALE = 1.4142135623730951


@functools.cache
def _build(B, D, k):
    chunk = 128 * k
    per_w = B // _NUM_WORKERS
    assert per_w % chunk == 0
    n_chunks = per_w // chunk
    rows_per_chunk = k

    mesh = plsc.VectorSubcoreMesh(core_axis_name="c", subcore_axis_name="s")

    @functools.partial(
        pl.kernel,
        mesh=mesh,
        compiler_params=pltpu.CompilerParams(use_tc_tiling_on_sc=False),
        out_type=jax.ShapeDtypeStruct((B // 128, 128, D), jnp.float32),
        scratch_types=[
            pltpu.VMEM((k, 128), jnp.int32),
            pltpu.VMEM((k, 128, D), jnp.float32),
            pltpu.VMEM((k, 128, D), jnp.float32),
            pltpu.SemaphoreType.DMA,
            pltpu.SemaphoreType.DMA,
        ],
    )
    def fused(x_hbm, wa_hbm, wb_hbm, out_hbm, idx_v, a_v, b_v, sem_a, sem_b):
        wid = lax.axis_index("s") * _NUM_CORES + lax.axis_index("c")
        row0 = wid * (per_w // 128)
        scale = jnp.float32(_SCALE)

        def compute_rows(j):
            @pl.loop(0, 128)
            def _(r):
                for h in range(D // _LANES):
                    sl = (j, r, pl.ds(h * _LANES, _LANES))
                    a_v[sl] = a_v[sl] + scale * b_v[sl]

        @pl.loop(0, n_chunks)
        def _(ci):
            row = row0 + ci * rows_per_chunk
            pltpu.sync_copy(x_hbm.at[pl.ds(row, rows_per_chunk)], idx_v)
            cps = [
                pltpu.async_copy(wa_hbm.at[idx_v.at[j]], a_v.at[j], sem_a)
                for j in range(k)
            ]
            cps += [
                pltpu.async_copy(wb_hbm.at[idx_v.at[j]], b_v.at[j], sem_b)
                for j in range(k)
            ]
            for cp in cps:
                cp.wait()
            for j in range(k):
                compute_rows(j)
            pltpu.sync_copy(a_v, out_hbm.at[pl.ds(row, rows_per_chunk)])

    return fused


@jax.jit
def kernel(x, Wa, Wb):
    B = x.size
    D = Wa.shape[1]
    x2d = x.reshape(B // 128, 128).astype(jnp.int32)
    out = _build(B, D, 4)(x2d, Wa, Wb)
    return out.reshape(x.shape + (D,))


# speedup vs baseline: 1.8667x; 1.2277x over previous
import functools

import jax
import jax.numpy as jnp
from jax import lax
from jax.experimental import pallas as pl
from jax.experimental.pallas import tpu as pltpu
from jax.experimental.pallas import tpu_sc as plsc

_NUM_CORES = 2
_NUM_SUBCORES = 16
_NUM_WORKERS = _NUM_CORES * _NUM_SUBCORES
_LANES = 16

_SCALE = 1.4142135623730951


@functools.cache
def _build(N, T, D, rows_per_chunk):
    per_w = N // _NUM_WORKERS
    n_chunks = per_w // rows_per_chunk
    assert per_w % rows_per_chunk == 0
    R = rows_per_chunk

    mesh = plsc.VectorSubcoreMesh(core_axis_name="c", subcore_axis_name="s")

    @functools.partial(
        pl.kernel,
        mesh=mesh,
        compiler_params=pltpu.CompilerParams(use_tc_tiling_on_sc=False),
        out_type=jax.ShapeDtypeStruct((N, T, D), jnp.float32),
        scratch_types=[
            pltpu.VMEM((R, T), jnp.int32),
            pltpu.VMEM((R, T, D), jnp.float32),
            pltpu.VMEM((R, T, D), jnp.float32),
            pltpu.SemaphoreType.DMA,
            pltpu.SemaphoreType.DMA,
        ],
    )
    def fused(x_hbm, wa_hbm, wb_hbm, out_hbm, idx_v, a_v, b_v, sem_a, sem_b):
        wid = lax.axis_index("s") * _NUM_CORES + lax.axis_index("c")
        row0 = wid * per_w
        scale = jnp.float32(_SCALE)

        def compute_rows(j):
            @pl.loop(0, T)
            def _(r):
                for h in range(D // _LANES):
                    sl = (j, r, pl.ds(h * _LANES, _LANES))
                    a_v[sl] = a_v[sl] + scale * b_v[sl]

        @pl.loop(0, n_chunks)
        def _(ci):
            row = row0 + ci * R
            pltpu.sync_copy(x_hbm.at[pl.ds(row, R)], idx_v)
            cps = [
                pltpu.async_copy(wa_hbm.at[idx_v.at[j]], a_v.at[j], sem_a)
                for j in range(R)
            ]
            cps += [
                pltpu.async_copy(wb_hbm.at[idx_v.at[j]], b_v.at[j], sem_b)
                for j in range(R)
            ]
            for cp in cps:
                cp.wait()
            for j in range(R):
                compute_rows(j)
            pltpu.sync_copy(a_v, out_hbm.at[pl.ds(row, R)])

    return fused


@jax.jit
def kernel(x, Wa, Wb):
    N, T = x.shape
    D = Wa.shape[1]
    return _build(N, T, D, 8)(x.astype(jnp.int32), Wa, Wb)
